# Initial kernel scaffold; baseline (speedup 1.0000x reference)
import jax
import jax.numpy as jnp
from jax.experimental import pallas as pl


def kernel(x, edge_index, W_l, W_r, att, bias):
    raise NotImplementedError("write your pallas kernel here")


# speedup vs baseline: 11.4699x; 11.4699x over previous
import functools

import jax
import jax.numpy as jnp
from jax import lax
from jax.experimental import pallas as pl
from jax.experimental.pallas import tpu as pltpu
from jax.experimental.pallas import tpu_sc as plsc

_N = 10000
_E = 320000
_D = 128
_C = 128
_L = 16
_NPAD = 10240
_ROWS_PT = _NPAD // 16
_K = 128
_NCH = _E // _K
_NW = 32
_NODE_BLK = 80
_E_BLK = 2000


def _dense_transforms(x, W_l, W_r):
    n, d = x.shape
    c = W_l.shape[1]
    blk = 400

    def body(x_ref, wl_ref, wr_ref, xl_ref, xr_ref):
        xb = x_ref[...]
        xl_ref[...] = jnp.dot(xb, wl_ref[...], preferred_element_type=jnp.float32)
        xr_ref[...] = jnp.dot(xb, wr_ref[...], preferred_element_type=jnp.float32)

    return pl.pallas_call(
        body,
        grid=(n // blk,),
        in_specs=[
            pl.BlockSpec((blk, d), lambda i: (i, 0)),
            pl.BlockSpec((d, c), lambda i: (0, 0)),
            pl.BlockSpec((d, c), lambda i: (0, 0)),
        ],
        out_specs=[
            pl.BlockSpec((blk, c), lambda i: (i, 0)),
            pl.BlockSpec((blk, c), lambda i: (i, 0)),
        ],
        out_shape=[
            jax.ShapeDtypeStruct((n, c), jnp.float32),
            jax.ShapeDtypeStruct((n, c), jnp.float32),
        ],
    )(x, W_l, W_r)


def _pass1_body(xl_hbm, xr_hbm, src_hbm, dst_hbm, att_hbm,
                num_hbm, den_hbm, w_hbm,
                xlb, xrb, srcb, dstb, wb, attv, zsmall, zbuf,
                num_sh, den_sh):
    cid = lax.axis_index("c")
    sid = lax.axis_index("s")
    wid = sid * 2 + cid
    zero16 = jnp.zeros((_L,), jnp.float32)

    def _z1(i, _):
        zsmall[pl.ds(i * _L, _L)] = zero16
        return 0
    lax.fori_loop(0, _ROWS_PT // _L, _z1, 0)

    def _z2(r, _):
        for cc in range(8):
            zbuf[r, pl.ds(cc * _L, _L)] = zero16
        return 0
    lax.fori_loop(0, 64, _z2, 0)

    for j in range(10):
        pltpu.sync_copy(zbuf, num_sh.at[pl.ds(sid * _ROWS_PT + j * 64, 64)])
    pltpu.sync_copy(zsmall, den_sh.at[pl.ds(sid * _ROWS_PT, _ROWS_PT)])
    plsc.subcore_barrier()

    pltpu.sync_copy(att_hbm, attv)
    att_regs = [attv[pl.ds(cc * _L, _L)] for cc in range(8)]
    iota16 = lax.iota(jnp.int32, _L)
    perms = [iota16 ^ k for k in (1, 2, 4, 8)]

    def _hsum(v):
        for p in perms:
            v = v + v[p]
        return v

    nbase = _NCH // _NW
    rem = _NCH - nbase * _NW
    my_n = nbase + jnp.where(wid < rem, 1, 0)

    def _chunk(ci, _):
        ebase = (wid + ci * _NW) * _K
        pltpu.sync_copy(src_hbm.at[pl.ds(ebase, _K)], srcb)
        pltpu.sync_copy(dst_hbm.at[pl.ds(ebase, _K)], dstb)
        pltpu.sync_copy(xl_hbm.at[srcb], xlb)
        pltpu.sync_copy(xr_hbm.at[dstb], xrb)

        def _score(g, _):
            svec = zero16
            for i in range(_L):
                e = g * _L + i
                acc = zero16
                for cc in range(8):
                    sl = pl.ds(cc * _L, _L)
                    v = xlb[e, sl] + xrb[e, sl]
                    acc = acc + att_regs[cc] * jnp.maximum(v, 0.2 * v)
                svec = jnp.where(iota16 == i, _hsum(acc), svec)
            wb[pl.ds(g * _L, _L)] = jnp.exp(svec)
            return 0
        lax.fori_loop(0, _K // _L, _score, 0)

        pltpu.sync_copy(wb, den_sh.at[dstb], add=True)

        def _scale(g, _):
            w16 = wb[pl.ds(g * _L, _L)]
            for i in range(_L):
                e = g * _L + i
                w = w16[i]
                for cc in range(8):
                    sl = pl.ds(cc * _L, _L)
                    xlb[e, sl] = xlb[e, sl] * w
            return 0
        lax.fori_loop(0, _K // _L, _scale, 0)

        pltpu.sync_copy(xlb, num_sh.at[dstb], add=True)
        pltpu.sync_copy(wb, w_hbm.at[pl.ds(ebase, _K)])
        return 0
    lax.fori_loop(0, my_n, _chunk, 0)

    plsc.subcore_barrier()
    col = pl.ds(sid * _ROWS_PT, _ROWS_PT)
    pltpu.sync_copy(den_sh.at[col], den_hbm.at[cid, col])
    for j in range(10):
        sl = pl.ds(sid * _ROWS_PT + j * 64, 64)
        pltpu.sync_copy(num_sh.at[sl], num_hbm.at[cid, sl])


def _pass2_body(num_hbm, den_hbm, dst_hbm, w_hbm, bias_hbm,
                out_hbm, alpha_hbm,
                denv, dtmp, biasv, dstb, wb, ab, n0, n1):
    cid = lax.axis_index("c")
    sid = lax.axis_index("s")
    wid = sid * 2 + cid

    pltpu.sync_copy(den_hbm.at[0], denv)
    pltpu.sync_copy(den_hbm.at[1], dtmp)

    def _dadd(i, _):
        sl = pl.ds(i * _L, _L)
        denv[sl] = denv[sl] + dtmp[sl]
        return 0
    lax.fori_loop(0, _NPAD // _L, _dadd, 0)

    pltpu.sync_copy(bias_hbm, biasv)
    bias_regs = [biasv[pl.ds(cc * _L, _L)] for cc in range(8)]
    iota16 = lax.iota(jnp.int32, _L)
    zero16 = jnp.zeros((_L,), jnp.float32)

    for j in range(_E // _NW // _E_BLK):
        ebase = wid * (_E // _NW) + j * _E_BLK
        pltpu.sync_copy(dst_hbm.at[pl.ds(ebase, _E_BLK)], dstb)
        pltpu.sync_copy(w_hbm.at[pl.ds(ebase, _E_BLK)], wb)

        def _alpha(i, _):
            sl = pl.ds(i * _L, _L)
            d16 = dstb[sl]
            dvals = zero16
            for k in range(_L):
                dk = denv[pl.ds(d16[k], _L)]
                dvals = jnp.where(iota16 == k, dk[0], dvals)
            ab[sl] = wb[sl] / dvals
            return 0
        lax.fori_loop(0, _E_BLK // _L, _alpha, 0)
        pltpu.sync_copy(ab, alpha_hbm.at[pl.ds(ebase, _E_BLK)])

    def _node_block(rbase):
        pltpu.sync_copy(num_hbm.at[0, pl.ds(rbase, _NODE_BLK)], n0)
        pltpu.sync_copy(num_hbm.at[1, pl.ds(rbase, _NODE_BLK)], n1)

        def _rowg(g, _):
            d16 = denv[pl.ds(rbase + g * _L, _L)]
            inv16 = 1.0 / jnp.maximum(d16, 1e-30)
            for i in range(_L):
                r = g * _L + i
                inv = inv16[i]
                for cc in range(8):
                    sl = pl.ds(cc * _L, _L)
                    n0[r, sl] = (n0[r, sl] + n1[r, sl]) * inv + bias_regs[cc]
            return 0
        lax.fori_loop(0, _NODE_BLK // _L, _rowg, 0)
        pltpu.sync_copy(n0, out_hbm.at[pl.ds(rbase, _NODE_BLK)])

    nblocks = _N // _NODE_BLK
    for k in range((nblocks + _NW - 1) // _NW):
        bid = wid + k * _NW

        @pl.when(bid < nblocks)
        def _blk(bid=bid):
            _node_block(bid * _NODE_BLK)


_mesh = plsc.VectorSubcoreMesh(core_axis_name="c", subcore_axis_name="s")

_pass1 = pl.kernel(
    _pass1_body,
    [
        jax.ShapeDtypeStruct((2, _NPAD, _C), jnp.float32),
        jax.ShapeDtypeStruct((2, _NPAD), jnp.float32),
        jax.ShapeDtypeStruct((_E,), jnp.float32),
    ],
    mesh=_mesh,
    scratch_types=[
        pltpu.VMEM((_K, _C), jnp.float32),
        pltpu.VMEM((_K, _C), jnp.float32),
        pltpu.VMEM((_K,), jnp.int32),
        pltpu.VMEM((_K,), jnp.int32),
        pltpu.VMEM((_K,), jnp.float32),
        pltpu.VMEM((_C,), jnp.float32),
        pltpu.VMEM((_ROWS_PT,), jnp.float32),
        pltpu.VMEM((64, _C), jnp.float32),
        pltpu.VMEM_SHARED((_NPAD, _C), jnp.float32),
        pltpu.VMEM_SHARED((_NPAD,), jnp.float32),
    ],
)

_pass2 = pl.kernel(
    _pass2_body,
    [
        jax.ShapeDtypeStruct((_N, _C), jnp.float32),
        jax.ShapeDtypeStruct((_E,), jnp.float32),
    ],
    mesh=_mesh,
    scratch_types=[
        pltpu.VMEM((_NPAD,), jnp.float32),
        pltpu.VMEM((_NPAD,), jnp.float32),
        pltpu.VMEM((_C,), jnp.float32),
        pltpu.VMEM((_E_BLK,), jnp.int32),
        pltpu.VMEM((_E_BLK,), jnp.float32),
        pltpu.VMEM((_E_BLK,), jnp.float32),
        pltpu.VMEM((_NODE_BLK, _C), jnp.float32),
        pltpu.VMEM((_NODE_BLK, _C), jnp.float32),
    ],
)


def kernel(x, edge_index, W_l, W_r, att, bias):
    xl, xr = _dense_transforms(x, W_l, W_r)
    src = edge_index[0]
    dst = edge_index[1]
    num, den, w = _pass1(xl, xr, src, dst, att.reshape(-1))
    out, alpha = _pass2(num, den, dst, w, bias)
    return out, alpha.reshape(_E, 1)


# speedup vs baseline: 14.3464x; 1.2508x over previous
import functools

import jax
import jax.numpy as jnp
from jax import lax
from jax.experimental import pallas as pl
from jax.experimental.pallas import tpu as pltpu
from jax.experimental.pallas import tpu_sc as plsc

_N = 10000
_E = 320000
_D = 128
_C = 128
_L = 16
_NPAD = 10240
_ROWS_PT = _NPAD // 16
_K = 64
_NCH = _E // _K
_NW = 32
_NODE_BLK = 80
_E_BLK = 2000


def _dense_transforms(x, W_l, W_r):
    n, d = x.shape
    c = W_l.shape[1]
    blk = 400

    def body(x_ref, wl_ref, wr_ref, xl_ref, xr_ref):
        xb = x_ref[...]
        xl_ref[...] = jnp.dot(xb, wl_ref[...], preferred_element_type=jnp.float32)
        xr_ref[...] = jnp.dot(xb, wr_ref[...], preferred_element_type=jnp.float32)

    return pl.pallas_call(
        body,
        grid=(n // blk,),
        in_specs=[
            pl.BlockSpec((blk, d), lambda i: (i, 0)),
            pl.BlockSpec((d, c), lambda i: (0, 0)),
            pl.BlockSpec((d, c), lambda i: (0, 0)),
        ],
        out_specs=[
            pl.BlockSpec((blk, c), lambda i: (i, 0)),
            pl.BlockSpec((blk, c), lambda i: (i, 0)),
        ],
        out_shape=[
            jax.ShapeDtypeStruct((n, c), jnp.float32),
            jax.ShapeDtypeStruct((n, c), jnp.float32),
        ],
    )(x, W_l, W_r)


def _pass1_body(xl_hbm, xr_hbm, src_hbm, dst_hbm, att_hbm,
                num_hbm, den_hbm, w_hbm,
                xlb0, xlb1, xrb0, xrb1, srcb0, srcb1, dstb0, dstb1, wb0, wb1,
                attv, zsmall, zbuf, num_sh, den_sh,
                gsem0, gsem1, ssem0, ssem1):
    xlb = (xlb0, xlb1)
    xrb = (xrb0, xrb1)
    srcb = (srcb0, srcb1)
    dstb = (dstb0, dstb1)
    wb = (wb0, wb1)
    gsem = (gsem0, gsem1)
    ssem = (ssem0, ssem1)
    cid = lax.axis_index("c")
    sid = lax.axis_index("s")
    wid = sid * 2 + cid
    zero16 = jnp.zeros((_L,), jnp.float32)

    def _z1(i, _):
        zsmall[pl.ds(i * _L, _L)] = zero16
        return 0
    lax.fori_loop(0, _ROWS_PT // _L, _z1, 0)

    def _z2(r, _):
        for cc in range(8):
            zbuf[r, pl.ds(cc * _L, _L)] = zero16
        return 0
    lax.fori_loop(0, 32, _z2, 0)

    for j in range(20):
        pltpu.sync_copy(zbuf, num_sh.at[pl.ds(sid * _ROWS_PT + j * 32, 32)])
    pltpu.sync_copy(zsmall, den_sh.at[pl.ds(sid * _ROWS_PT, _ROWS_PT)])
    plsc.subcore_barrier()

    pltpu.sync_copy(att_hbm, attv)
    att_regs = [attv[pl.ds(cc * _L, _L)] for cc in range(8)]
    iota16 = lax.iota(jnp.int32, _L)
    perms = [iota16 ^ k for k in (1, 2, 4, 8)]

    def _hsum(v):
        for p in perms:
            v = v + v[p]
        return v

    ncw = _NCH // _NW
    rem = _NCH - ncw * _NW

    def _ebase(i):
        return (wid + i * _NW) * _K

    def _fetch(i, s):
        eb = _ebase(i)
        pltpu.sync_copy(src_hbm.at[pl.ds(eb, _K)], srcb[s])
        pltpu.sync_copy(dst_hbm.at[pl.ds(eb, _K)], dstb[s])
        pltpu.async_copy(xl_hbm.at[srcb[s]], xlb[s], gsem[s])
        pltpu.async_copy(xr_hbm.at[dstb[s]], xrb[s], gsem[s])

    def _wait_gathers(s):
        pltpu.make_async_copy(xl_hbm.at[srcb[s]], xlb[s], gsem[s]).wait()
        pltpu.make_async_copy(xr_hbm.at[dstb[s]], xrb[s], gsem[s]).wait()

    def _issue_scatters(i, s):
        eb = _ebase(i)
        pltpu.sync_copy(xlb[s], num_sh.at[dstb[s]], add=True)
        pltpu.sync_copy(wb[s], den_sh.at[dstb[s]], add=True)
        pltpu.async_copy(wb[s], w_hbm.at[pl.ds(eb, _K)], ssem[s])

    def _wait_scatters(i, s):
        eb = _ebase(i)
        pltpu.make_async_copy(wb[s], w_hbm.at[pl.ds(eb, _K)], ssem[s]).wait()

    def _compute(s):
        def _score(g, _):
            svec = zero16
            for i in range(_L):
                e = g * _L + i
                acc = zero16
                for cc in range(8):
                    sl = pl.ds(cc * _L, _L)
                    v = xlb[s][e, sl] + xrb[s][e, sl]
                    acc = acc + att_regs[cc] * jnp.maximum(v, 0.2 * v)
                svec = jnp.where(iota16 == i, _hsum(acc), svec)
            wb[s][pl.ds(g * _L, _L)] = jnp.exp(svec)
            return 0
        lax.fori_loop(0, _K // _L, _score, 0)

        def _scale(g, _):
            w16 = wb[s][pl.ds(g * _L, _L)]
            for i in range(_L):
                e = g * _L + i
                w = w16[i]
                for cc in range(8):
                    sl = pl.ds(cc * _L, _L)
                    xlb[s][e, sl] = xlb[s][e, sl] * w
            return 0
        lax.fori_loop(0, _K // _L, _scale, 0)

    _fetch(0, 0)

    def _pair(p, _):
        for b in (0, 1):
            cur = 2 * p + b

            @pl.when(cur + 1 < ncw)
            def _prefetch():
                @pl.when(cur >= 1)
                def _drain():
                    _wait_scatters(cur - 1, 1 - b)
                _fetch(cur + 1, 1 - b)

            _wait_gathers(b)
            _compute(b)
            _issue_scatters(cur, b)
        return 0
    lax.fori_loop(0, ncw // 2, _pair, 0)
    _wait_scatters(ncw - 2, 0)
    _wait_scatters(ncw - 1, 1)

    @pl.when(wid < rem)
    def _tail():
        eb = _ebase(ncw)
        pltpu.sync_copy(src_hbm.at[pl.ds(eb, _K)], srcb[0])
        pltpu.sync_copy(dst_hbm.at[pl.ds(eb, _K)], dstb[0])
        pltpu.sync_copy(xl_hbm.at[srcb[0]], xlb[0])
        pltpu.sync_copy(xr_hbm.at[dstb[0]], xrb[0])
        _compute(0)
        pltpu.sync_copy(xlb[0], num_sh.at[dstb[0]], add=True)
        pltpu.sync_copy(wb[0], den_sh.at[dstb[0]], add=True)
        pltpu.sync_copy(wb[0], w_hbm.at[pl.ds(eb, _K)])

    plsc.subcore_barrier()
    col = pl.ds(sid * _ROWS_PT, _ROWS_PT)
    pltpu.sync_copy(den_sh.at[col], den_hbm.at[cid, col])
    for j in range(10):
        sl = pl.ds(sid * _ROWS_PT + j * 64, 64)
        pltpu.sync_copy(num_sh.at[sl], num_hbm.at[cid, sl])


def _pass2_body(num_hbm, den_hbm, dst_hbm, w_hbm, bias_hbm,
                out_hbm, alpha_hbm,
                denv, dtmp, biasv, dstb, wb, ab, n0, n1):
    cid = lax.axis_index("c")
    sid = lax.axis_index("s")
    wid = sid * 2 + cid

    pltpu.sync_copy(den_hbm.at[0], denv)
    pltpu.sync_copy(den_hbm.at[1], dtmp)

    def _dadd(i, _):
        sl = pl.ds(i * _L, _L)
        denv[sl] = denv[sl] + dtmp[sl]
        return 0
    lax.fori_loop(0, _NPAD // _L, _dadd, 0)

    pltpu.sync_copy(bias_hbm, biasv)
    bias_regs = [biasv[pl.ds(cc * _L, _L)] for cc in range(8)]
    iota16 = lax.iota(jnp.int32, _L)
    zero16 = jnp.zeros((_L,), jnp.float32)

    for j in range(_E // _NW // _E_BLK):
        ebase = wid * (_E // _NW) + j * _E_BLK
        pltpu.sync_copy(dst_hbm.at[pl.ds(ebase, _E_BLK)], dstb)
        pltpu.sync_copy(w_hbm.at[pl.ds(ebase, _E_BLK)], wb)

        def _alpha(i, _):
            sl = pl.ds(i * _L, _L)
            d16 = dstb[sl]
            dvals = zero16
            for k in range(_L):
                dk = denv[pl.ds(d16[k], _L)]
                dvals = jnp.where(iota16 == k, dk[0], dvals)
            ab[sl] = wb[sl] / dvals
            return 0
        lax.fori_loop(0, _E_BLK // _L, _alpha, 0)
        pltpu.sync_copy(ab, alpha_hbm.at[pl.ds(ebase, _E_BLK)])

    def _node_block(rbase):
        pltpu.sync_copy(num_hbm.at[0, pl.ds(rbase, _NODE_BLK)], n0)
        pltpu.sync_copy(num_hbm.at[1, pl.ds(rbase, _NODE_BLK)], n1)

        def _rowg(g, _):
            d16 = denv[pl.ds(rbase + g * _L, _L)]
            inv16 = 1.0 / jnp.maximum(d16, 1e-30)
            for i in range(_L):
                r = g * _L + i
                inv = inv16[i]
                for cc in range(8):
                    sl = pl.ds(cc * _L, _L)
                    n0[r, sl] = (n0[r, sl] + n1[r, sl]) * inv + bias_regs[cc]
            return 0
        lax.fori_loop(0, _NODE_BLK // _L, _rowg, 0)
        pltpu.sync_copy(n0, out_hbm.at[pl.ds(rbase, _NODE_BLK)])

    nblocks = _N // _NODE_BLK
    for k in range((nblocks + _NW - 1) // _NW):
        bid = wid + k * _NW

        @pl.when(bid < nblocks)
        def _blk(bid=bid):
            _node_block(bid * _NODE_BLK)


_mesh = plsc.VectorSubcoreMesh(core_axis_name="c", subcore_axis_name="s")

_pass1 = pl.kernel(
    _pass1_body,
    [
        jax.ShapeDtypeStruct((2, _NPAD, _C), jnp.float32),
        jax.ShapeDtypeStruct((2, _NPAD), jnp.float32),
        jax.ShapeDtypeStruct((_E,), jnp.float32),
    ],
    mesh=_mesh,
    scratch_types=[
        pltpu.VMEM((_K, _C), jnp.float32),
        pltpu.VMEM((_K, _C), jnp.float32),
        pltpu.VMEM((_K, _C), jnp.float32),
        pltpu.VMEM((_K, _C), jnp.float32),
        pltpu.VMEM((_K,), jnp.int32),
        pltpu.VMEM((_K,), jnp.int32),
        pltpu.VMEM((_K,), jnp.int32),
        pltpu.VMEM((_K,), jnp.int32),
        pltpu.VMEM((_K,), jnp.float32),
        pltpu.VMEM((_K,), jnp.float32),
        pltpu.VMEM((_C,), jnp.float32),
        pltpu.VMEM((_ROWS_PT,), jnp.float32),
        pltpu.VMEM((32, _C), jnp.float32),
        pltpu.VMEM_SHARED((_NPAD, _C), jnp.float32),
        pltpu.VMEM_SHARED((_NPAD,), jnp.float32),
        pltpu.SemaphoreType.DMA,
        pltpu.SemaphoreType.DMA,
        pltpu.SemaphoreType.DMA,
        pltpu.SemaphoreType.DMA,
    ],
)

_pass2 = pl.kernel(
    _pass2_body,
    [
        jax.ShapeDtypeStruct((_N, _C), jnp.float32),
        jax.ShapeDtypeStruct((_E,), jnp.float32),
    ],
    mesh=_mesh,
    scratch_types=[
        pltpu.VMEM((_NPAD,), jnp.float32),
        pltpu.VMEM((_NPAD,), jnp.float32),
        pltpu.VMEM((_C,), jnp.float32),
        pltpu.VMEM((_E_BLK,), jnp.int32),
        pltpu.VMEM((_E_BLK,), jnp.float32),
        pltpu.VMEM((_E_BLK,), jnp.float32),
        pltpu.VMEM((_NODE_BLK, _C), jnp.float32),
        pltpu.VMEM((_NODE_BLK, _C), jnp.float32),
    ],
)


def kernel(x, edge_index, W_l, W_r, att, bias):
    xl, xr = _dense_transforms(x, W_l, W_r)
    src = edge_index[0]
    dst = edge_index[1]
    num, den, w = _pass1(xl, xr, src, dst, att.reshape(-1))
    out, alpha = _pass2(num, den, dst, w, bias)
    return out, alpha.reshape(_E, 1)


# speedup vs baseline: 14.9520x; 1.0422x over previous
import functools

import jax
import jax.numpy as jnp
from jax import lax
from jax.experimental import pallas as pl
from jax.experimental.pallas import tpu as pltpu
from jax.experimental.pallas import tpu_sc as plsc

_N = 10000
_E = 320000
_D = 128
_C = 128
_L = 16
_NPAD = 10240
_ROWS_PT = _NPAD // 16
_K = 64
_NCH = _E // _K
_NW = 32
_NODE_BLK = 80
_E_BLK = 2000


def _dense_transforms(x, W_l, W_r):
    n, d = x.shape
    c = W_l.shape[1]
    blk = 400

    def body(x_ref, wl_ref, wr_ref, xl_ref, xr_ref):
        xb = x_ref[...]
        xl_ref[...] = jnp.dot(xb, wl_ref[...], preferred_element_type=jnp.float32)
        xr_ref[...] = jnp.dot(xb, wr_ref[...], preferred_element_type=jnp.float32)

    return pl.pallas_call(
        body,
        grid=(n // blk,),
        in_specs=[
            pl.BlockSpec((blk, d), lambda i: (i, 0)),
            pl.BlockSpec((d, c), lambda i: (0, 0)),
            pl.BlockSpec((d, c), lambda i: (0, 0)),
        ],
        out_specs=[
            pl.BlockSpec((blk, c), lambda i: (i, 0)),
            pl.BlockSpec((blk, c), lambda i: (i, 0)),
        ],
        out_shape=[
            jax.ShapeDtypeStruct((n, c), jnp.float32),
            jax.ShapeDtypeStruct((n, c), jnp.float32),
        ],
    )(x, W_l, W_r)


def _pass1_body(xl_hbm, xr_hbm, src_hbm, dst_hbm, att_hbm,
                num_hbm, den_hbm, w_hbm,
                xlb0, xlb1, xrb0, xrb1, srcb0, srcb1, dstb0, dstb1, wb0, wb1,
                attv, zsmall, zbuf, num_sh, den_sh,
                gsem0, gsem1, nsem0, nsem1, dsem0, dsem1, wsem0, wsem1):
    xlb = (xlb0, xlb1)
    xrb = (xrb0, xrb1)
    srcb = (srcb0, srcb1)
    dstb = (dstb0, dstb1)
    wb = (wb0, wb1)
    gsem = (gsem0, gsem1)
    nsem = (nsem0, nsem1)
    dsem = (dsem0, dsem1)
    wsem = (wsem0, wsem1)
    cid = lax.axis_index("c")
    sid = lax.axis_index("s")
    wid = sid * 2 + cid
    zero16 = jnp.zeros((_L,), jnp.float32)

    def _z1(i, _):
        zsmall[pl.ds(i * _L, _L)] = zero16
        return 0
    lax.fori_loop(0, _ROWS_PT // _L, _z1, 0)

    def _z2(r, _):
        for cc in range(8):
            zbuf[r, pl.ds(cc * _L, _L)] = zero16
        return 0
    lax.fori_loop(0, 32, _z2, 0)

    for j in range(20):
        pltpu.sync_copy(zbuf, num_sh.at[pl.ds(sid * _ROWS_PT + j * 32, 32)])
    pltpu.sync_copy(zsmall, den_sh.at[pl.ds(sid * _ROWS_PT, _ROWS_PT)])
    plsc.subcore_barrier()

    pltpu.sync_copy(att_hbm, attv)
    att_regs = [attv[pl.ds(cc * _L, _L)] for cc in range(8)]
    iota16 = lax.iota(jnp.int32, _L)
    perms = [iota16 ^ k for k in (1, 2, 4, 8)]

    def _hsum(v):
        for p in perms:
            v = v + v[p]
        return v

    ncw = _NCH // _NW
    rem = _NCH - ncw * _NW

    def _ebase(i):
        return (wid + i * _NW) * _K

    def _fetch(i, s):
        eb = _ebase(i)
        pltpu.sync_copy(src_hbm.at[pl.ds(eb, _K)], srcb[s])
        pltpu.sync_copy(dst_hbm.at[pl.ds(eb, _K)], dstb[s])
        pltpu.async_copy(xl_hbm.at[srcb[s]], xlb[s], gsem[s])
        pltpu.async_copy(xr_hbm.at[dstb[s]], xrb[s], gsem[s])

    def _wait_gathers(s):
        pltpu.make_async_copy(xl_hbm.at[srcb[s]], xlb[s], gsem[s]).wait()
        pltpu.make_async_copy(xr_hbm.at[dstb[s]], xrb[s], gsem[s]).wait()

    def _issue_scatters(i, s):
        eb = _ebase(i)
        pltpu.async_copy(xlb[s], num_sh.at[dstb[s]], nsem[s], add=True)
        pltpu.async_copy(wb[s], den_sh.at[dstb[s]], dsem[s], add=True)
        pltpu.async_copy(wb[s], w_hbm.at[pl.ds(eb, _K)], wsem[s])

    def _wait_scatters(i, s):
        eb = _ebase(i)
        pltpu.make_async_copy(xlb[s], num_sh.at[dstb[s]], nsem[s]).wait()
        pltpu.make_async_copy(wb[s], den_sh.at[dstb[s]], dsem[s]).wait()
        pltpu.make_async_copy(wb[s], w_hbm.at[pl.ds(eb, _K)], wsem[s]).wait()

    def _compute(s):
        def _score(g, _):
            svec = zero16
            for i in range(_L):
                e = g * _L + i
                acc = zero16
                for cc in range(8):
                    sl = pl.ds(cc * _L, _L)
                    v = xlb[s][e, sl] + xrb[s][e, sl]
                    acc = acc + att_regs[cc] * jnp.maximum(v, 0.2 * v)
                svec = jnp.where(iota16 == i, _hsum(acc), svec)
            wb[s][pl.ds(g * _L, _L)] = jnp.exp(svec)
            return 0
        lax.fori_loop(0, _K // _L, _score, 0)

        def _scale(g, _):
            w16 = wb[s][pl.ds(g * _L, _L)]
            for i in range(_L):
                e = g * _L + i
                w = w16[i]
                for cc in range(8):
                    sl = pl.ds(cc * _L, _L)
                    xlb[s][e, sl] = xlb[s][e, sl] * w
            return 0
        lax.fori_loop(0, _K // _L, _scale, 0)

    _fetch(0, 0)

    def _pair(p, _):
        for b in (0, 1):
            cur = 2 * p + b

            @pl.when(cur + 1 < ncw)
            def _prefetch():
                @pl.when(cur >= 1)
                def _drain():
                    _wait_scatters(cur - 1, 1 - b)
                _fetch(cur + 1, 1 - b)

            _wait_gathers(b)
            _compute(b)
            _issue_scatters(cur, b)
        return 0
    lax.fori_loop(0, ncw // 2, _pair, 0)
    _wait_scatters(ncw - 2, 0)
    _wait_scatters(ncw - 1, 1)

    @pl.when(wid < rem)
    def _tail():
        eb = _ebase(ncw)
        pltpu.sync_copy(src_hbm.at[pl.ds(eb, _K)], srcb[0])
        pltpu.sync_copy(dst_hbm.at[pl.ds(eb, _K)], dstb[0])
        pltpu.sync_copy(xl_hbm.at[srcb[0]], xlb[0])
        pltpu.sync_copy(xr_hbm.at[dstb[0]], xrb[0])
        _compute(0)
        pltpu.sync_copy(xlb[0], num_sh.at[dstb[0]], add=True)
        pltpu.sync_copy(wb[0], den_sh.at[dstb[0]], add=True)
        pltpu.sync_copy(wb[0], w_hbm.at[pl.ds(eb, _K)])

    plsc.subcore_barrier()
    col = pl.ds(sid * _ROWS_PT, _ROWS_PT)
    pltpu.sync_copy(den_sh.at[col], den_hbm.at[cid, col])
    for j in range(10):
        sl = pl.ds(sid * _ROWS_PT + j * 64, 64)
        pltpu.sync_copy(num_sh.at[sl], num_hbm.at[cid, sl])


def _pass2_body(num_hbm, den_hbm, dst_hbm, w_hbm, bias_hbm,
                out_hbm, alpha_hbm,
                denv, dtmp, biasv, dstb, wb, ab, n0, n1):
    cid = lax.axis_index("c")
    sid = lax.axis_index("s")
    wid = sid * 2 + cid

    pltpu.sync_copy(den_hbm.at[0], denv)
    pltpu.sync_copy(den_hbm.at[1], dtmp)

    def _dadd(i, _):
        sl = pl.ds(i * _L, _L)
        denv[sl] = denv[sl] + dtmp[sl]
        return 0
    lax.fori_loop(0, _NPAD // _L, _dadd, 0)

    pltpu.sync_copy(bias_hbm, biasv)
    bias_regs = [biasv[pl.ds(cc * _L, _L)] for cc in range(8)]
    iota16 = lax.iota(jnp.int32, _L)
    zero16 = jnp.zeros((_L,), jnp.float32)

    for j in range(_E // _NW // _E_BLK):
        ebase = wid * (_E // _NW) + j * _E_BLK
        pltpu.sync_copy(dst_hbm.at[pl.ds(ebase, _E_BLK)], dstb)
        pltpu.sync_copy(w_hbm.at[pl.ds(ebase, _E_BLK)], wb)

        def _alpha(i, _):
            sl = pl.ds(i * _L, _L)
            d16 = dstb[sl]
            dvals = zero16
            for k in range(_L):
                dk = denv[pl.ds(d16[k], _L)]
                dvals = jnp.where(iota16 == k, dk[0], dvals)
            ab[sl] = wb[sl] / dvals
            return 0
        lax.fori_loop(0, _E_BLK // _L, _alpha, 0)
        pltpu.sync_copy(ab, alpha_hbm.at[pl.ds(ebase, _E_BLK)])

    def _node_block(rbase):
        pltpu.sync_copy(num_hbm.at[0, pl.ds(rbase, _NODE_BLK)], n0)
        pltpu.sync_copy(num_hbm.at[1, pl.ds(rbase, _NODE_BLK)], n1)

        def _rowg(g, _):
            d16 = denv[pl.ds(rbase + g * _L, _L)]
            inv16 = 1.0 / jnp.maximum(d16, 1e-30)
            for i in range(_L):
                r = g * _L + i
                inv = inv16[i]
                for cc in range(8):
                    sl = pl.ds(cc * _L, _L)
                    n0[r, sl] = (n0[r, sl] + n1[r, sl]) * inv + bias_regs[cc]
            return 0
        lax.fori_loop(0, _NODE_BLK // _L, _rowg, 0)
        pltpu.sync_copy(n0, out_hbm.at[pl.ds(rbase, _NODE_BLK)])

    nblocks = _N // _NODE_BLK
    for k in range((nblocks + _NW - 1) // _NW):
        bid = wid + k * _NW

        @pl.when(bid < nblocks)
        def _blk(bid=bid):
            _node_block(bid * _NODE_BLK)


_mesh = plsc.VectorSubcoreMesh(core_axis_name="c", subcore_axis_name="s")

_pass1 = pl.kernel(
    _pass1_body,
    [
        jax.ShapeDtypeStruct((2, _NPAD, _C), jnp.float32),
        jax.ShapeDtypeStruct((2, _NPAD), jnp.float32),
        jax.ShapeDtypeStruct((_E,), jnp.float32),
    ],
    mesh=_mesh,
    scratch_types=[
        pltpu.VMEM((_K, _C), jnp.float32),
        pltpu.VMEM((_K, _C), jnp.float32),
        pltpu.VMEM((_K, _C), jnp.float32),
        pltpu.VMEM((_K, _C), jnp.float32),
        pltpu.VMEM((_K,), jnp.int32),
        pltpu.VMEM((_K,), jnp.int32),
        pltpu.VMEM((_K,), jnp.int32),
        pltpu.VMEM((_K,), jnp.int32),
        pltpu.VMEM((_K,), jnp.float32),
        pltpu.VMEM((_K,), jnp.float32),
        pltpu.VMEM((_C,), jnp.float32),
        pltpu.VMEM((_ROWS_PT,), jnp.float32),
        pltpu.VMEM((32, _C), jnp.float32),
        pltpu.VMEM_SHARED((_NPAD, _C), jnp.float32),
        pltpu.VMEM_SHARED((_NPAD,), jnp.float32),
        pltpu.SemaphoreType.DMA,
        pltpu.SemaphoreType.DMA,
        pltpu.SemaphoreType.DMA,
        pltpu.SemaphoreType.DMA,
        pltpu.SemaphoreType.DMA,
        pltpu.SemaphoreType.DMA,
        pltpu.SemaphoreType.DMA,
        pltpu.SemaphoreType.DMA,
    ],
)

_pass2 = pl.kernel(
    _pass2_body,
    [
        jax.ShapeDtypeStruct((_N, _C), jnp.float32),
        jax.ShapeDtypeStruct((_E,), jnp.float32),
    ],
    mesh=_mesh,
    scratch_types=[
        pltpu.VMEM((_NPAD,), jnp.float32),
        pltpu.VMEM((_NPAD,), jnp.float32),
        pltpu.VMEM((_C,), jnp.float32),
        pltpu.VMEM((_E_BLK,), jnp.int32),
        pltpu.VMEM((_E_BLK,), jnp.float32),
        pltpu.VMEM((_E_BLK,), jnp.float32),
        pltpu.VMEM((_NODE_BLK, _C), jnp.float32),
        pltpu.VMEM((_NODE_BLK, _C), jnp.float32),
    ],
)


def kernel(x, edge_index, W_l, W_r, att, bias):
    xl, xr = _dense_transforms(x, W_l, W_r)
    src = edge_index[0]
    dst = edge_index[1]
    num, den, w = _pass1(xl, xr, src, dst, att.reshape(-1))
    out, alpha = _pass2(num, den, dst, w, bias)
    return out, alpha.reshape(_E, 1)


# speedup vs baseline: 18.6311x; 1.2461x over previous
import functools

import jax
import jax.numpy as jnp
from jax import lax
from jax.experimental import pallas as pl
from jax.experimental.pallas import tpu as pltpu
from jax.experimental.pallas import tpu_sc as plsc

_N = 10000
_E = 320000
_D = 128
_C = 128
_L = 16
_NPAD = 10240
_ROWS_PT = _NPAD // 16
_K = 64
_NCH = _E // _K
_NW = 32
_NODE_BLK = 80
_E_BLK = 2000


def _dense_transforms(x, W_l, W_r):
    n, d = x.shape
    c = W_l.shape[1]
    blk = 400

    def body(x_ref, wl_ref, wr_ref, xl_ref, xr_ref):
        xb = x_ref[...]
        xl_ref[...] = jnp.dot(xb, wl_ref[...], preferred_element_type=jnp.float32)
        xr_ref[...] = jnp.dot(xb, wr_ref[...], preferred_element_type=jnp.float32)

    return pl.pallas_call(
        body,
        grid=(n // blk,),
        in_specs=[
            pl.BlockSpec((blk, d), lambda i: (i, 0)),
            pl.BlockSpec((d, c), lambda i: (0, 0)),
            pl.BlockSpec((d, c), lambda i: (0, 0)),
        ],
        out_specs=[
            pl.BlockSpec((blk, c), lambda i: (i, 0)),
            pl.BlockSpec((blk, c), lambda i: (i, 0)),
        ],
        out_shape=[
            jax.ShapeDtypeStruct((n, c), jnp.float32),
            jax.ShapeDtypeStruct((n, c), jnp.float32),
        ],
    )(x, W_l, W_r)


def _pass1_body(xl_hbm, xr_hbm, src_hbm, dst_hbm, att_hbm,
                num_hbm, den_hbm, w_hbm,
                xlb0, xlb1, xrb0, xrb1,
                srcb0, srcb1, srcb2, srcb3, dstb0, dstb1, dstb2, dstb3,
                wb0, wb1,
                attv, zsmall, zbuf, num_sh, den_sh,
                gsem0, gsem1, nsem0, nsem1, dsem0, dsem1, wsem0, wsem1,
                isem0, isem1, isem2, isem3):
    xlb = (xlb0, xlb1)
    xrb = (xrb0, xrb1)
    srcb = (srcb0, srcb1, srcb2, srcb3)
    dstb = (dstb0, dstb1, dstb2, dstb3)
    wb = (wb0, wb1)
    gsem = (gsem0, gsem1)
    nsem = (nsem0, nsem1)
    dsem = (dsem0, dsem1)
    wsem = (wsem0, wsem1)
    isem = (isem0, isem1, isem2, isem3)
    cid = lax.axis_index("c")
    sid = lax.axis_index("s")
    wid = sid * 2 + cid
    zero16 = jnp.zeros((_L,), jnp.float32)

    def _z1(i, _):
        zsmall[pl.ds(i * _L, _L)] = zero16
        return 0
    lax.fori_loop(0, _ROWS_PT // _L, _z1, 0)

    def _z2(r, _):
        for cc in range(8):
            zbuf[r, pl.ds(cc * _L, _L)] = zero16
        return 0
    lax.fori_loop(0, 32, _z2, 0)

    for j in range(20):
        pltpu.sync_copy(zbuf, num_sh.at[pl.ds(sid * _ROWS_PT + j * 32, 32)])
    pltpu.sync_copy(zsmall, den_sh.at[pl.ds(sid * _ROWS_PT, _ROWS_PT)])
    plsc.subcore_barrier()

    pltpu.sync_copy(att_hbm, attv)
    att_regs = [attv[pl.ds(cc * _L, _L)] for cc in range(8)]
    iota16 = lax.iota(jnp.int32, _L)
    perms = [iota16 ^ k for k in (1, 2, 4, 8)]

    def _hsum(v):
        for p in perms:
            v = v + v[p]
        return v

    ncw = _NCH // _NW
    rem = _NCH - ncw * _NW

    def _ebase(i):
        return (wid + i * _NW) * _K

    def _issue_idx(i, q):
        eb = _ebase(i)
        pltpu.async_copy(src_hbm.at[pl.ds(eb, _K)], srcb[q], isem[q])
        pltpu.async_copy(dst_hbm.at[pl.ds(eb, _K)], dstb[q], isem[q])

    def _wait_idx(q):
        pltpu.make_async_copy(src_hbm.at[pl.ds(0, _K)], srcb[q], isem[q]).wait()
        pltpu.make_async_copy(dst_hbm.at[pl.ds(0, _K)], dstb[q], isem[q]).wait()

    def _issue_gathers(q, s):
        pltpu.async_copy(xl_hbm.at[srcb[q]], xlb[s], gsem[s])
        pltpu.async_copy(xr_hbm.at[dstb[q]], xrb[s], gsem[s])

    def _wait_gathers(q, s):
        pltpu.make_async_copy(xl_hbm.at[srcb[q]], xlb[s], gsem[s]).wait()
        pltpu.make_async_copy(xr_hbm.at[dstb[q]], xrb[s], gsem[s]).wait()

    def _issue_scatters(i, q, s):
        eb = _ebase(i)
        pltpu.async_copy(xlb[s], num_sh.at[dstb[q]], nsem[s], add=True)
        pltpu.async_copy(wb[s], den_sh.at[dstb[q]], dsem[s], add=True)
        pltpu.async_copy(wb[s], w_hbm.at[pl.ds(eb, _K)], wsem[s])

    def _wait_scatters(i, q, s):
        eb = _ebase(i)
        pltpu.make_async_copy(xlb[s], num_sh.at[dstb[q]], nsem[s]).wait()
        pltpu.make_async_copy(wb[s], den_sh.at[dstb[q]], dsem[s]).wait()
        pltpu.make_async_copy(wb[s], w_hbm.at[pl.ds(eb, _K)], wsem[s]).wait()

    def _compute(s):
        def _score(g, _):
            svec = zero16
            for i in range(_L):
                e = g * _L + i
                acc = zero16
                for cc in range(8):
                    sl = pl.ds(cc * _L, _L)
                    v = xlb[s][e, sl] + xrb[s][e, sl]
                    acc = acc + att_regs[cc] * jnp.maximum(v, 0.2 * v)
                svec = jnp.where(iota16 == i, _hsum(acc), svec)
            wb[s][pl.ds(g * _L, _L)] = jnp.exp(svec)
            return 0
        lax.fori_loop(0, _K // _L, _score, 0)

        def _scale(g, _):
            w16 = wb[s][pl.ds(g * _L, _L)]
            for i in range(_L):
                e = g * _L + i
                w = w16[i]
                for cc in range(8):
                    sl = pl.ds(cc * _L, _L)
                    xlb[s][e, sl] = xlb[s][e, sl] * w
            return 0
        lax.fori_loop(0, _K // _L, _scale, 0)

    _issue_idx(0, 0)
    _issue_idx(1, 1)
    _wait_idx(0)
    _issue_gathers(0, 0)

    def _quad(t, _):
        for u in range(4):
            cur = 4 * t + u
            s = u % 2

            @pl.when(cur + 2 < ncw)
            def _pi():
                _issue_idx(cur + 2, (u + 2) % 4)

            @pl.when(cur + 1 < ncw)
            def _pf():
                @pl.when(cur >= 1)
                def _dr():
                    _wait_scatters(cur - 1, (u + 3) % 4, 1 - s)
                _wait_idx((u + 1) % 4)
                _issue_gathers((u + 1) % 4, 1 - s)

            _wait_gathers(u, s)
            _compute(s)
            _issue_scatters(cur, u, s)
        return 0
    lax.fori_loop(0, ncw // 4, _quad, 0)
    _wait_scatters(ncw - 2, (ncw - 2) % 4, 0)
    _wait_scatters(ncw - 1, (ncw - 1) % 4, 1)

    @pl.when(wid < rem)
    def _tail():
        eb = _ebase(ncw)
        pltpu.sync_copy(src_hbm.at[pl.ds(eb, _K)], srcb[0])
        pltpu.sync_copy(dst_hbm.at[pl.ds(eb, _K)], dstb[0])
        pltpu.sync_copy(xl_hbm.at[srcb[0]], xlb[0])
        pltpu.sync_copy(xr_hbm.at[dstb[0]], xrb[0])
        _compute(0)
        pltpu.sync_copy(xlb[0], num_sh.at[dstb[0]], add=True)
        pltpu.sync_copy(wb[0], den_sh.at[dstb[0]], add=True)
        pltpu.sync_copy(wb[0], w_hbm.at[pl.ds(eb, _K)])

    plsc.subcore_barrier()
    col = pl.ds(sid * _ROWS_PT, _ROWS_PT)
    pltpu.sync_copy(den_sh.at[col], den_hbm.at[cid, col])
    for j in range(10):
        sl = pl.ds(sid * _ROWS_PT + j * 64, 64)
        pltpu.sync_copy(num_sh.at[sl], num_hbm.at[cid, sl])


def _pass2_body(num_hbm, den_hbm, dst_hbm, w_hbm, bias_hbm,
                out_hbm, alpha_hbm,
                denv, dtmp, biasv, dstb, wb, ab, n0, n1):
    cid = lax.axis_index("c")
    sid = lax.axis_index("s")
    wid = sid * 2 + cid

    pltpu.sync_copy(den_hbm.at[0], denv)
    pltpu.sync_copy(den_hbm.at[1], dtmp)

    def _dadd(i, _):
        sl = pl.ds(i * _L, _L)
        denv[sl] = denv[sl] + dtmp[sl]
        return 0
    lax.fori_loop(0, _NPAD // _L, _dadd, 0)

    pltpu.sync_copy(bias_hbm, biasv)
    bias_regs = [biasv[pl.ds(cc * _L, _L)] for cc in range(8)]
    iota16 = lax.iota(jnp.int32, _L)
    zero16 = jnp.zeros((_L,), jnp.float32)

    for j in range(_E // _NW // _E_BLK):
        ebase = wid * (_E // _NW) + j * _E_BLK
        pltpu.sync_copy(dst_hbm.at[pl.ds(ebase, _E_BLK)], dstb)
        pltpu.sync_copy(w_hbm.at[pl.ds(ebase, _E_BLK)], wb)

        def _alpha(i, _):
            sl = pl.ds(i * _L, _L)
            d16 = dstb[sl]
            dvals = zero16
            for k in range(_L):
                dk = denv[pl.ds(d16[k], _L)]
                dvals = jnp.where(iota16 == k, dk[0], dvals)
            ab[sl] = wb[sl] / dvals
            return 0
        lax.fori_loop(0, _E_BLK // _L, _alpha, 0)
        pltpu.sync_copy(ab, alpha_hbm.at[pl.ds(ebase, _E_BLK)])

    def _node_block(rbase):
        pltpu.sync_copy(num_hbm.at[0, pl.ds(rbase, _NODE_BLK)], n0)
        pltpu.sync_copy(num_hbm.at[1, pl.ds(rbase, _NODE_BLK)], n1)

        def _rowg(g, _):
            d16 = denv[pl.ds(rbase + g * _L, _L)]
            inv16 = 1.0 / jnp.maximum(d16, 1e-30)
            for i in range(_L):
                r = g * _L + i
                inv = inv16[i]
                for cc in range(8):
                    sl = pl.ds(cc * _L, _L)
                    n0[r, sl] = (n0[r, sl] + n1[r, sl]) * inv + bias_regs[cc]
            return 0
        lax.fori_loop(0, _NODE_BLK // _L, _rowg, 0)
        pltpu.sync_copy(n0, out_hbm.at[pl.ds(rbase, _NODE_BLK)])

    nblocks = _N // _NODE_BLK
    for k in range((nblocks + _NW - 1) // _NW):
        bid = wid + k * _NW

        @pl.when(bid < nblocks)
        def _blk(bid=bid):
            _node_block(bid * _NODE_BLK)


_mesh = plsc.VectorSubcoreMesh(core_axis_name="c", subcore_axis_name="s")

_pass1 = pl.kernel(
    _pass1_body,
    [
        jax.ShapeDtypeStruct((2, _NPAD, _C), jnp.float32),
        jax.ShapeDtypeStruct((2, _NPAD), jnp.float32),
        jax.ShapeDtypeStruct((_E,), jnp.float32),
    ],
    mesh=_mesh,
    scratch_types=[
        pltpu.VMEM((_K, _C), jnp.float32),
        pltpu.VMEM((_K, _C), jnp.float32),
        pltpu.VMEM((_K, _C), jnp.float32),
        pltpu.VMEM((_K, _C), jnp.float32),
        pltpu.VMEM((_K,), jnp.int32),
        pltpu.VMEM((_K,), jnp.int32),
        pltpu.VMEM((_K,), jnp.int32),
        pltpu.VMEM((_K,), jnp.int32),
        pltpu.VMEM((_K,), jnp.int32),
        pltpu.VMEM((_K,), jnp.int32),
        pltpu.VMEM((_K,), jnp.int32),
        pltpu.VMEM((_K,), jnp.int32),
        pltpu.VMEM((_K,), jnp.float32),
        pltpu.VMEM((_K,), jnp.float32),
        pltpu.VMEM((_C,), jnp.float32),
        pltpu.VMEM((_ROWS_PT,), jnp.float32),
        pltpu.VMEM((32, _C), jnp.float32),
        pltpu.VMEM_SHARED((_NPAD, _C), jnp.float32),
        pltpu.VMEM_SHARED((_NPAD,), jnp.float32),
        pltpu.SemaphoreType.DMA,
        pltpu.SemaphoreType.DMA,
        pltpu.SemaphoreType.DMA,
        pltpu.SemaphoreType.DMA,
        pltpu.SemaphoreType.DMA,
        pltpu.SemaphoreType.DMA,
        pltpu.SemaphoreType.DMA,
        pltpu.SemaphoreType.DMA,
        pltpu.SemaphoreType.DMA,
        pltpu.SemaphoreType.DMA,
        pltpu.SemaphoreType.DMA,
        pltpu.SemaphoreType.DMA,
    ],
)

_pass2 = pl.kernel(
    _pass2_body,
    [
        jax.ShapeDtypeStruct((_N, _C), jnp.float32),
        jax.ShapeDtypeStruct((_E,), jnp.float32),
    ],
    mesh=_mesh,
    scratch_types=[
        pltpu.VMEM((_NPAD,), jnp.float32),
        pltpu.VMEM((_NPAD,), jnp.float32),
        pltpu.VMEM((_C,), jnp.float32),
        pltpu.VMEM((_E_BLK,), jnp.int32),
        pltpu.VMEM((_E_BLK,), jnp.float32),
        pltpu.VMEM((_E_BLK,), jnp.float32),
        pltpu.VMEM((_NODE_BLK, _C), jnp.float32),
        pltpu.VMEM((_NODE_BLK, _C), jnp.float32),
    ],
)


def kernel(x, edge_index, W_l, W_r, att, bias):
    xl, xr = _dense_transforms(x, W_l, W_r)
    src = edge_index[0]
    dst = edge_index[1]
    num, den, w = _pass1(xl, xr, src, dst, att.reshape(-1))
    out, alpha = _pass2(num, den, dst, w, bias)
    return out, alpha.reshape(_E, 1)


# speedup vs baseline: 19.2834x; 1.0350x over previous
import functools

import jax
import jax.numpy as jnp
from jax import lax
from jax.experimental import pallas as pl
from jax.experimental.pallas import tpu as pltpu
from jax.experimental.pallas import tpu_sc as plsc

_N = 10000
_E = 320000
_D = 128
_C = 128
_L = 16
_NPAD = 10240
_ROWS_PT = _NPAD // 16
_K = 64
_NCH = _E // _K
_NW = 32
_NODE_BLK = 80
_E_BLK = 2000


def _dense_transforms(x, W_l, W_r):
    n, d = x.shape
    c = W_l.shape[1]
    blk = 400

    def body(x_ref, wl_ref, wr_ref, xl_ref, xr_ref):
        xb = x_ref[...]
        xl_ref[...] = jnp.dot(xb, wl_ref[...], preferred_element_type=jnp.float32)
        xr_ref[...] = jnp.dot(xb, wr_ref[...], preferred_element_type=jnp.float32)

    return pl.pallas_call(
        body,
        grid=(n // blk,),
        in_specs=[
            pl.BlockSpec((blk, d), lambda i: (i, 0)),
            pl.BlockSpec((d, c), lambda i: (0, 0)),
            pl.BlockSpec((d, c), lambda i: (0, 0)),
        ],
        out_specs=[
            pl.BlockSpec((blk, c), lambda i: (i, 0)),
            pl.BlockSpec((blk, c), lambda i: (i, 0)),
        ],
        out_shape=[
            jax.ShapeDtypeStruct((n, c), jnp.float32),
            jax.ShapeDtypeStruct((n, c), jnp.float32),
        ],
    )(x, W_l, W_r)


def _pass1_body(xl_hbm, xr_hbm, src_hbm, dst_hbm, att_hbm,
                num_hbm, den_hbm, w_hbm,
                xlb0, xlb1, xrb0, xrb1,
                srcb0, srcb1, srcb2, srcb3, dstb0, dstb1, dstb2, dstb3,
                wb0, wb1,
                attv, zsmall, zbuf, num_sh, den_sh,
                gsem0, gsem1, nsem0, nsem1, dsem0, dsem1, wsem0, wsem1,
                isem0, isem1, isem2, isem3):
    xlb = (xlb0, xlb1)
    xrb = (xrb0, xrb1)
    srcb = (srcb0, srcb1, srcb2, srcb3)
    dstb = (dstb0, dstb1, dstb2, dstb3)
    wb = (wb0, wb1)
    gsem = (gsem0, gsem1)
    nsem = (nsem0, nsem1)
    dsem = (dsem0, dsem1)
    wsem = (wsem0, wsem1)
    isem = (isem0, isem1, isem2, isem3)
    cid = lax.axis_index("c")
    sid = lax.axis_index("s")
    wid = sid * 2 + cid
    zero16 = jnp.zeros((_L,), jnp.float32)

    def _z1(i, _):
        zsmall[pl.ds(i * _L, _L)] = zero16
        return 0
    lax.fori_loop(0, _ROWS_PT // _L, _z1, 0)

    def _z2(r, _):
        for cc in range(8):
            zbuf[r, pl.ds(cc * _L, _L)] = zero16
        return 0
    lax.fori_loop(0, 32, _z2, 0)

    for j in range(20):
        pltpu.sync_copy(zbuf, num_sh.at[pl.ds(sid * _ROWS_PT + j * 32, 32)])
    pltpu.sync_copy(zsmall, den_sh.at[pl.ds(sid * _ROWS_PT, _ROWS_PT)])
    plsc.subcore_barrier()

    pltpu.sync_copy(att_hbm, attv)
    att_regs = [attv[pl.ds(cc * _L, _L)] for cc in range(8)]
    iota16 = lax.iota(jnp.int32, _L)
    perms = [iota16 ^ k for k in (1, 2, 4, 8)]

    def _hsum(v):
        for p in perms:
            v = v + v[p]
        return v

    ncw = _NCH // _NW
    rem = _NCH - ncw * _NW

    def _ebase(i):
        return (wid + i * _NW) * _K

    def _issue_idx(i, q):
        eb = _ebase(i)
        pltpu.async_copy(src_hbm.at[pl.ds(eb, _K)], srcb[q], isem[q])
        pltpu.async_copy(dst_hbm.at[pl.ds(eb, _K)], dstb[q], isem[q])

    def _wait_idx(q):
        pltpu.make_async_copy(src_hbm.at[pl.ds(0, _K)], srcb[q], isem[q]).wait()
        pltpu.make_async_copy(dst_hbm.at[pl.ds(0, _K)], dstb[q], isem[q]).wait()

    def _issue_gathers(q, s):
        pltpu.async_copy(xl_hbm.at[srcb[q]], xlb[s], gsem[s])
        pltpu.async_copy(xr_hbm.at[dstb[q]], xrb[s], gsem[s])

    def _wait_gathers(q, s):
        pltpu.make_async_copy(xl_hbm.at[srcb[q]], xlb[s], gsem[s]).wait()
        pltpu.make_async_copy(xr_hbm.at[dstb[q]], xrb[s], gsem[s]).wait()

    def _issue_scatters(i, q, s):
        eb = _ebase(i)
        pltpu.async_copy(xlb[s], num_sh.at[dstb[q]], nsem[s], add=True)
        pltpu.async_copy(wb[s], den_sh.at[dstb[q]], dsem[s], add=True)
        pltpu.async_copy(wb[s], w_hbm.at[pl.ds(eb, _K)], wsem[s])

    def _wait_scatters(i, q, s):
        eb = _ebase(i)
        pltpu.make_async_copy(xlb[s], num_sh.at[dstb[q]], nsem[s]).wait()
        pltpu.make_async_copy(wb[s], den_sh.at[dstb[q]], dsem[s]).wait()
        pltpu.make_async_copy(wb[s], w_hbm.at[pl.ds(eb, _K)], wsem[s]).wait()

    def _compute(s):
        def _score(g, _):
            svec = zero16
            for i in range(_L):
                e = g * _L + i
                acc = zero16
                for cc in range(8):
                    sl = pl.ds(cc * _L, _L)
                    v = xlb[s][e, sl] + xrb[s][e, sl]
                    acc = acc + att_regs[cc] * jnp.maximum(v, 0.2 * v)
                svec = jnp.where(iota16 == i, _hsum(acc), svec)
            wb[s][pl.ds(g * _L, _L)] = jnp.exp(svec)
            return 0
        lax.fori_loop(0, _K // _L, _score, 0)

        def _scale(g, _):
            w16 = wb[s][pl.ds(g * _L, _L)]
            for i in range(_L):
                e = g * _L + i
                w = w16[i]
                for cc in range(8):
                    sl = pl.ds(cc * _L, _L)
                    xlb[s][e, sl] = xlb[s][e, sl] * w
            return 0
        lax.fori_loop(0, _K // _L, _scale, 0)

    _issue_idx(0, 0)
    _issue_idx(1, 1)
    _wait_idx(0)
    _issue_gathers(0, 0)

    def _quad(t, _):
        for u in range(4):
            cur = 4 * t + u
            s = u % 2

            @pl.when(cur + 2 < ncw)
            def _pi():
                _issue_idx(cur + 2, (u + 2) % 4)

            @pl.when(cur + 1 < ncw)
            def _pf():
                @pl.when(cur >= 1)
                def _dr():
                    _wait_scatters(cur - 1, (u + 3) % 4, 1 - s)
                _wait_idx((u + 1) % 4)
                _issue_gathers((u + 1) % 4, 1 - s)

            _wait_gathers(u, s)
            _compute(s)
            _issue_scatters(cur, u, s)
        return 0
    lax.fori_loop(0, ncw // 4, _quad, 0)
    _wait_scatters(ncw - 2, (ncw - 2) % 4, 0)
    _wait_scatters(ncw - 1, (ncw - 1) % 4, 1)

    @pl.when(wid < rem)
    def _tail():
        eb = _ebase(ncw)
        pltpu.sync_copy(src_hbm.at[pl.ds(eb, _K)], srcb[0])
        pltpu.sync_copy(dst_hbm.at[pl.ds(eb, _K)], dstb[0])
        pltpu.sync_copy(xl_hbm.at[srcb[0]], xlb[0])
        pltpu.sync_copy(xr_hbm.at[dstb[0]], xrb[0])
        _compute(0)
        pltpu.sync_copy(xlb[0], num_sh.at[dstb[0]], add=True)
        pltpu.sync_copy(wb[0], den_sh.at[dstb[0]], add=True)
        pltpu.sync_copy(wb[0], w_hbm.at[pl.ds(eb, _K)])

    plsc.subcore_barrier()
    col = pl.ds(sid * _ROWS_PT, _ROWS_PT)
    pltpu.sync_copy(den_sh.at[col], den_hbm.at[cid, col])
    for j in range(10):
        sl = pl.ds(sid * _ROWS_PT + j * 64, 64)
        pltpu.sync_copy(num_sh.at[sl], num_hbm.at[cid, sl])


def _pass2_body(num_hbm, den_hbm, dst_hbm, w_hbm, bias_hbm,
                out_hbm, alpha_hbm,
                denv, dtmp, biasv,
                dstb0, dstb1, wbb0, wbb1, ab0, ab1,
                n00, n01, n10, n11,
                dsem, esem0, esem1, asem0, asem1,
                nsem0, nsem1, osem0, osem1):
    cid = lax.axis_index("c")
    sid = lax.axis_index("s")
    wid = sid * 2 + cid
    dstb = (dstb0, dstb1)
    wbb = (wbb0, wbb1)
    ab = (ab0, ab1)
    n0 = (n00, n01)
    n1 = (n10, n11)
    esem = (esem0, esem1)
    asem = (asem0, asem1)
    nsem = (nsem0, nsem1)
    osem = (osem0, osem1)

    nebs = _E // _NW // _E_BLK
    nblocks = _N // _NODE_BLK
    nkb = (nblocks + _NW - 1) // _NW

    def _eb(j):
        return wid * (_E // _NW) + j * _E_BLK

    def _issue_eloads(j, s):
        pltpu.async_copy(dst_hbm.at[pl.ds(_eb(j), _E_BLK)], dstb[s], esem[s])
        pltpu.async_copy(w_hbm.at[pl.ds(_eb(j), _E_BLK)], wbb[s], esem[s])

    def _wait_eloads(s):
        pltpu.make_async_copy(dst_hbm.at[pl.ds(0, _E_BLK)], dstb[s], esem[s]).wait()
        pltpu.make_async_copy(w_hbm.at[pl.ds(0, _E_BLK)], wbb[s], esem[s]).wait()

    def _rbase(k):
        return (wid + k * _NW) * _NODE_BLK

    def _issue_nloads(k, m):
        rb = _rbase(k)
        pltpu.async_copy(num_hbm.at[0, pl.ds(rb, _NODE_BLK)], n0[m], nsem[m])
        pltpu.async_copy(num_hbm.at[1, pl.ds(rb, _NODE_BLK)], n1[m], nsem[m])

    def _wait_nloads(m):
        pltpu.make_async_copy(num_hbm.at[0, pl.ds(0, _NODE_BLK)], n0[m], nsem[m]).wait()
        pltpu.make_async_copy(num_hbm.at[1, pl.ds(0, _NODE_BLK)], n1[m], nsem[m]).wait()

    pltpu.async_copy(den_hbm.at[0], denv, dsem)
    pltpu.async_copy(den_hbm.at[1], dtmp, dsem)
    pltpu.async_copy(bias_hbm, biasv, dsem)
    _issue_eloads(0, 0)
    _issue_nloads(0, 0)

    pltpu.make_async_copy(den_hbm.at[0], denv, dsem).wait()
    pltpu.make_async_copy(den_hbm.at[1], dtmp, dsem).wait()
    pltpu.make_async_copy(bias_hbm, biasv, dsem).wait()

    def _dadd(i, _):
        sl = pl.ds(i * _L, _L)
        denv[sl] = denv[sl] + dtmp[sl]
        return 0
    lax.fori_loop(0, _NPAD // _L, _dadd, 0)

    bias_regs = [biasv[pl.ds(cc * _L, _L)] for cc in range(8)]
    iota16 = lax.iota(jnp.int32, _L)
    zero16 = jnp.zeros((_L,), jnp.float32)

    for j in range(nebs):
        s = j % 2
        if j + 1 < nebs:
            _issue_eloads(j + 1, 1 - s)
        _wait_eloads(s)
        if j >= 2:
            pltpu.make_async_copy(ab[s], alpha_hbm.at[pl.ds(0, _E_BLK)],
                                  asem[s]).wait()

        def _alpha(i, _, s=s):
            sl = pl.ds(i * _L, _L)
            d16 = dstb[s][sl]
            dvals = zero16
            for k in range(_L):
                dk = denv[pl.ds(d16[k], _L)]
                dvals = jnp.where(iota16 == k, dk[0], dvals)
            ab[s][sl] = wbb[s][sl] / dvals
            return 0
        lax.fori_loop(0, _E_BLK // _L, _alpha, 0)
        pltpu.async_copy(ab[s], alpha_hbm.at[pl.ds(_eb(j), _E_BLK)], asem[s])
    for j in (nebs - 2, nebs - 1):
        s = j % 2
        pltpu.make_async_copy(ab[s], alpha_hbm.at[pl.ds(0, _E_BLK)],
                              asem[s]).wait()

    for k in range(nkb):
        m = k % 2
        if k >= 1:
            @pl.when(_rbase(k - 1) < _N)
            def _dro(m=m):
                pltpu.make_async_copy(n0[1 - m],
                                      out_hbm.at[pl.ds(0, _NODE_BLK)],
                                      osem[1 - m]).wait()
        if k + 1 < nkb:
            @pl.when(_rbase(k + 1) < _N)
            def _pfn(k=k, m=m):
                _issue_nloads(k + 1, 1 - m)

        @pl.when(_rbase(k) < _N)
        def _blk(k=k, m=m):
            rb = _rbase(k)
            _wait_nloads(m)

            def _rowg(g, _):
                d16 = denv[pl.ds(rb + g * _L, _L)]
                inv16 = 1.0 / jnp.maximum(d16, 1e-30)
                for i in range(_L):
                    r = g * _L + i
                    inv = inv16[i]
                    for cc in range(8):
                        sl = pl.ds(cc * _L, _L)
                        n0[m][r, sl] = ((n0[m][r, sl] + n1[m][r, sl]) * inv
                                        + bias_regs[cc])
                return 0
            lax.fori_loop(0, _NODE_BLK // _L, _rowg, 0)
            pltpu.async_copy(n0[m], out_hbm.at[pl.ds(rb, _NODE_BLK)], osem[m])
    @pl.when(_rbase(nkb - 1) < _N)
    def _drl():
        pltpu.make_async_copy(n0[(nkb - 1) % 2],
                              out_hbm.at[pl.ds(0, _NODE_BLK)],
                              osem[(nkb - 1) % 2]).wait()


_mesh = plsc.VectorSubcoreMesh(core_axis_name="c", subcore_axis_name="s")

_pass1 = pl.kernel(
    _pass1_body,
    [
        jax.ShapeDtypeStruct((2, _NPAD, _C), jnp.float32),
        jax.ShapeDtypeStruct((2, _NPAD), jnp.float32),
        jax.ShapeDtypeStruct((_E,), jnp.float32),
    ],
    mesh=_mesh,
    scratch_types=[
        pltpu.VMEM((_K, _C), jnp.float32),
        pltpu.VMEM((_K, _C), jnp.float32),
        pltpu.VMEM((_K, _C), jnp.float32),
        pltpu.VMEM((_K, _C), jnp.float32),
        pltpu.VMEM((_K,), jnp.int32),
        pltpu.VMEM((_K,), jnp.int32),
        pltpu.VMEM((_K,), jnp.int32),
        pltpu.VMEM((_K,), jnp.int32),
        pltpu.VMEM((_K,), jnp.int32),
        pltpu.VMEM((_K,), jnp.int32),
        pltpu.VMEM((_K,), jnp.int32),
        pltpu.VMEM((_K,), jnp.int32),
        pltpu.VMEM((_K,), jnp.float32),
        pltpu.VMEM((_K,), jnp.float32),
        pltpu.VMEM((_C,), jnp.float32),
        pltpu.VMEM((_ROWS_PT,), jnp.float32),
        pltpu.VMEM((32, _C), jnp.float32),
        pltpu.VMEM_SHARED((_NPAD, _C), jnp.float32),
        pltpu.VMEM_SHARED((_NPAD,), jnp.float32),
        pltpu.SemaphoreType.DMA,
        pltpu.SemaphoreType.DMA,
        pltpu.SemaphoreType.DMA,
        pltpu.SemaphoreType.DMA,
        pltpu.SemaphoreType.DMA,
        pltpu.SemaphoreType.DMA,
        pltpu.SemaphoreType.DMA,
        pltpu.SemaphoreType.DMA,
        pltpu.SemaphoreType.DMA,
        pltpu.SemaphoreType.DMA,
        pltpu.SemaphoreType.DMA,
        pltpu.SemaphoreType.DMA,
    ],
)

_pass2 = pl.kernel(
    _pass2_body,
    [
        jax.ShapeDtypeStruct((_N, _C), jnp.float32),
        jax.ShapeDtypeStruct((_E,), jnp.float32),
    ],
    mesh=_mesh,
    scratch_types=[
        pltpu.VMEM((_NPAD,), jnp.float32),
        pltpu.VMEM((_NPAD,), jnp.float32),
        pltpu.VMEM((_C,), jnp.float32),
        pltpu.VMEM((_E_BLK,), jnp.int32),
        pltpu.VMEM((_E_BLK,), jnp.int32),
        pltpu.VMEM((_E_BLK,), jnp.float32),
        pltpu.VMEM((_E_BLK,), jnp.float32),
        pltpu.VMEM((_E_BLK,), jnp.float32),
        pltpu.VMEM((_E_BLK,), jnp.float32),
        pltpu.VMEM((_NODE_BLK, _C), jnp.float32),
        pltpu.VMEM((_NODE_BLK, _C), jnp.float32),
        pltpu.VMEM((_NODE_BLK, _C), jnp.float32),
        pltpu.VMEM((_NODE_BLK, _C), jnp.float32),
        pltpu.SemaphoreType.DMA,
        pltpu.SemaphoreType.DMA,
        pltpu.SemaphoreType.DMA,
        pltpu.SemaphoreType.DMA,
        pltpu.SemaphoreType.DMA,
        pltpu.SemaphoreType.DMA,
        pltpu.SemaphoreType.DMA,
        pltpu.SemaphoreType.DMA,
        pltpu.SemaphoreType.DMA,
    ],
)


def kernel(x, edge_index, W_l, W_r, att, bias):
    xl, xr = _dense_transforms(x, W_l, W_r)
    src = edge_index[0]
    dst = edge_index[1]
    num, den, w = _pass1(xl, xr, src, dst, att.reshape(-1))
    out, alpha = _pass2(num, den, dst, w, bias)
    return out, alpha.reshape(_E, 1)


# speedup vs baseline: 20.1939x; 1.0472x over previous
import functools

import jax
import jax.numpy as jnp
from jax import lax
from jax.experimental import pallas as pl
from jax.experimental.pallas import tpu as pltpu
from jax.experimental.pallas import tpu_sc as plsc

_N = 10000
_E = 320000
_D = 128
_C = 128
_L = 16
_NPAD = 10240
_ROWS_PT = _NPAD // 16
_K = 64
_NCH = _E // _K
_NW = 32
_NODE_BLK = 80
_E_BLK = 2000


def _dense_transforms(x, W_l, W_r):
    n, d = x.shape
    c = W_l.shape[1]
    blk = 400

    def body(x_ref, wl_ref, wr_ref, xl_ref, xr_ref):
        xb = x_ref[...]
        xl_ref[...] = jnp.dot(xb, wl_ref[...], preferred_element_type=jnp.float32)
        xr_ref[...] = jnp.dot(xb, wr_ref[...], preferred_element_type=jnp.float32)

    return pl.pallas_call(
        body,
        grid=(n // blk,),
        in_specs=[
            pl.BlockSpec((blk, d), lambda i: (i, 0)),
            pl.BlockSpec((d, c), lambda i: (0, 0)),
            pl.BlockSpec((d, c), lambda i: (0, 0)),
        ],
        out_specs=[
            pl.BlockSpec((blk, c), lambda i: (i, 0)),
            pl.BlockSpec((blk, c), lambda i: (i, 0)),
        ],
        out_shape=[
            jax.ShapeDtypeStruct((n, c), jnp.float32),
            jax.ShapeDtypeStruct((n, c), jnp.float32),
        ],
    )(x, W_l, W_r)


def _pass1_body(xl_hbm, xr_hbm, ei_hbm, att_hbm,
                num_hbm, den_hbm, w_hbm,
                xlb0, xlb1, xrb0, xrb1,
                srcb0, srcb1, srcb2, srcb3, dstb0, dstb1, dstb2, dstb3,
                wb0, wb1,
                attv, zsmall, zbuf, num_sh, den_sh,
                gsem0, gsem1, nsem0, nsem1, dsem0, dsem1, wsem0, wsem1,
                isem0, isem1, isem2, isem3):
    xlb = (xlb0, xlb1)
    xrb = (xrb0, xrb1)
    srcb = (srcb0, srcb1, srcb2, srcb3)
    dstb = (dstb0, dstb1, dstb2, dstb3)
    wb = (wb0, wb1)
    gsem = (gsem0, gsem1)
    nsem = (nsem0, nsem1)
    dsem = (dsem0, dsem1)
    wsem = (wsem0, wsem1)
    isem = (isem0, isem1, isem2, isem3)
    cid = lax.axis_index("c")
    sid = lax.axis_index("s")
    wid = sid * 2 + cid
    zero16 = jnp.zeros((_L,), jnp.float32)

    def _z1(i, _):
        zsmall[pl.ds(i * _L, _L)] = zero16
        return 0
    lax.fori_loop(0, _ROWS_PT // _L, _z1, 0)

    def _z2(r, _):
        for cc in range(8):
            zbuf[r, pl.ds(cc * _L, _L)] = zero16
        return 0
    lax.fori_loop(0, 32, _z2, 0)

    for j in range(20):
        pltpu.sync_copy(zbuf, num_sh.at[pl.ds(sid * _ROWS_PT + j * 32, 32)])
    pltpu.sync_copy(zsmall, den_sh.at[pl.ds(sid * _ROWS_PT, _ROWS_PT)])
    plsc.subcore_barrier()

    pltpu.sync_copy(att_hbm, attv)
    att_regs = [attv[pl.ds(cc * _L, _L)] for cc in range(8)]
    iota16 = lax.iota(jnp.int32, _L)
    perms = [iota16 ^ k for k in (1, 2, 4, 8)]

    def _hsum(v):
        for p in perms:
            v = v + v[p]
        return v

    ncw = _NCH // _NW
    rem = _NCH - ncw * _NW

    def _ebase(i):
        return (wid + i * _NW) * _K

    def _issue_idx(i, q):
        eb = _ebase(i)
        pltpu.async_copy(ei_hbm.at[pl.ds(eb, _K)], srcb[q], isem[q])
        pltpu.async_copy(ei_hbm.at[pl.ds(_E + eb, _K)], dstb[q], isem[q])

    def _wait_idx(q):
        pltpu.make_async_copy(ei_hbm.at[pl.ds(0, _K)], srcb[q], isem[q]).wait()
        pltpu.make_async_copy(ei_hbm.at[pl.ds(0, _K)], dstb[q], isem[q]).wait()

    def _issue_gathers(q, s):
        pltpu.async_copy(xl_hbm.at[srcb[q]], xlb[s], gsem[s])
        pltpu.async_copy(xr_hbm.at[dstb[q]], xrb[s], gsem[s])

    def _wait_gathers(q, s):
        pltpu.make_async_copy(xl_hbm.at[srcb[q]], xlb[s], gsem[s]).wait()
        pltpu.make_async_copy(xr_hbm.at[dstb[q]], xrb[s], gsem[s]).wait()

    def _issue_scatters(i, q, s):
        eb = _ebase(i)
        pltpu.async_copy(xlb[s], num_sh.at[dstb[q]], nsem[s], add=True)
        pltpu.async_copy(wb[s], den_sh.at[dstb[q]], dsem[s], add=True)
        pltpu.async_copy(wb[s], w_hbm.at[pl.ds(eb, _K)], wsem[s])

    def _wait_scatters(i, q, s):
        eb = _ebase(i)
        pltpu.make_async_copy(xlb[s], num_sh.at[dstb[q]], nsem[s]).wait()
        pltpu.make_async_copy(wb[s], den_sh.at[dstb[q]], dsem[s]).wait()
        pltpu.make_async_copy(wb[s], w_hbm.at[pl.ds(eb, _K)], wsem[s]).wait()

    def _compute(s):
        def _score(g, _):
            svec = zero16
            for i in range(_L):
                e = g * _L + i
                acc = zero16
                for cc in range(8):
                    sl = pl.ds(cc * _L, _L)
                    v = xlb[s][e, sl] + xrb[s][e, sl]
                    acc = acc + att_regs[cc] * jnp.maximum(v, 0.2 * v)
                svec = jnp.where(iota16 == i, _hsum(acc), svec)
            wb[s][pl.ds(g * _L, _L)] = jnp.exp(svec)
            return 0
        lax.fori_loop(0, _K // _L, _score, 0)

        def _scale(g, _):
            w16 = wb[s][pl.ds(g * _L, _L)]
            for i in range(_L):
                e = g * _L + i
                w = w16[i]
                for cc in range(8):
                    sl = pl.ds(cc * _L, _L)
                    xlb[s][e, sl] = xlb[s][e, sl] * w
            return 0
        lax.fori_loop(0, _K // _L, _scale, 0)

    _issue_idx(0, 0)
    _issue_idx(1, 1)
    _wait_idx(0)
    _issue_gathers(0, 0)

    def _quad(t, _):
        for u in range(4):
            cur = 4 * t + u
            s = u % 2

            @pl.when(cur + 2 < ncw)
            def _pi():
                _issue_idx(cur + 2, (u + 2) % 4)

            @pl.when(cur + 1 < ncw)
            def _pf():
                @pl.when(cur >= 1)
                def _dr():
                    _wait_scatters(cur - 1, (u + 3) % 4, 1 - s)
                _wait_idx((u + 1) % 4)
                _issue_gathers((u + 1) % 4, 1 - s)

            _wait_gathers(u, s)
            _compute(s)
            _issue_scatters(cur, u, s)
        return 0
    lax.fori_loop(0, ncw // 4, _quad, 0)
    _wait_scatters(ncw - 2, (ncw - 2) % 4, 0)
    _wait_scatters(ncw - 1, (ncw - 1) % 4, 1)

    @pl.when(wid < rem)
    def _tail():
        eb = _ebase(ncw)
        pltpu.sync_copy(ei_hbm.at[pl.ds(eb, _K)], srcb[0])
        pltpu.sync_copy(ei_hbm.at[pl.ds(_E + eb, _K)], dstb[0])
        pltpu.sync_copy(xl_hbm.at[srcb[0]], xlb[0])
        pltpu.sync_copy(xr_hbm.at[dstb[0]], xrb[0])
        _compute(0)
        pltpu.sync_copy(xlb[0], num_sh.at[dstb[0]], add=True)
        pltpu.sync_copy(wb[0], den_sh.at[dstb[0]], add=True)
        pltpu.sync_copy(wb[0], w_hbm.at[pl.ds(eb, _K)])

    plsc.subcore_barrier()
    col = pl.ds(sid * _ROWS_PT, _ROWS_PT)
    pltpu.sync_copy(den_sh.at[col], den_hbm.at[cid, col])
    for j in range(10):
        sl = pl.ds(sid * _ROWS_PT + j * 64, 64)
        pltpu.sync_copy(num_sh.at[sl], num_hbm.at[cid, sl])


def _pass2_body(num_hbm, den_hbm, ei_hbm, w_hbm, bias_hbm,
                out_hbm, alpha_hbm,
                denv, dtmp, biasv,
                dstb0, dstb1, wbb0, wbb1, ab0, ab1,
                n00, n01, n10, n11,
                dsem, esem0, esem1, asem0, asem1,
                nsem0, nsem1, osem0, osem1):
    cid = lax.axis_index("c")
    sid = lax.axis_index("s")
    wid = sid * 2 + cid
    dstb = (dstb0, dstb1)
    wbb = (wbb0, wbb1)
    ab = (ab0, ab1)
    n0 = (n00, n01)
    n1 = (n10, n11)
    esem = (esem0, esem1)
    asem = (asem0, asem1)
    nsem = (nsem0, nsem1)
    osem = (osem0, osem1)

    nebs = _E // _NW // _E_BLK
    nblocks = _N // _NODE_BLK
    nkb = (nblocks + _NW - 1) // _NW

    def _eb(j):
        return wid * (_E // _NW) + j * _E_BLK

    def _issue_eloads(j, s):
        pltpu.async_copy(ei_hbm.at[pl.ds(_E + _eb(j), _E_BLK)], dstb[s], esem[s])
        pltpu.async_copy(w_hbm.at[pl.ds(_eb(j), _E_BLK)], wbb[s], esem[s])

    def _wait_eloads(s):
        pltpu.make_async_copy(ei_hbm.at[pl.ds(0, _E_BLK)], dstb[s], esem[s]).wait()
        pltpu.make_async_copy(w_hbm.at[pl.ds(0, _E_BLK)], wbb[s], esem[s]).wait()

    def _rbase(k):
        return (wid + k * _NW) * _NODE_BLK

    def _issue_nloads(k, m):
        rb = _rbase(k)
        pltpu.async_copy(num_hbm.at[0, pl.ds(rb, _NODE_BLK)], n0[m], nsem[m])
        pltpu.async_copy(num_hbm.at[1, pl.ds(rb, _NODE_BLK)], n1[m], nsem[m])

    def _wait_nloads(m):
        pltpu.make_async_copy(num_hbm.at[0, pl.ds(0, _NODE_BLK)], n0[m], nsem[m]).wait()
        pltpu.make_async_copy(num_hbm.at[1, pl.ds(0, _NODE_BLK)], n1[m], nsem[m]).wait()

    pltpu.async_copy(den_hbm.at[0], denv, dsem)
    pltpu.async_copy(den_hbm.at[1], dtmp, dsem)
    pltpu.async_copy(bias_hbm, biasv, dsem)
    _issue_eloads(0, 0)
    _issue_nloads(0, 0)

    pltpu.make_async_copy(den_hbm.at[0], denv, dsem).wait()
    pltpu.make_async_copy(den_hbm.at[1], dtmp, dsem).wait()
    pltpu.make_async_copy(bias_hbm, biasv, dsem).wait()

    def _dadd(i, _):
        sl = pl.ds(i * _L, _L)
        denv[sl] = denv[sl] + dtmp[sl]
        return 0
    lax.fori_loop(0, _NPAD // _L, _dadd, 0)

    bias_regs = [biasv[pl.ds(cc * _L, _L)] for cc in range(8)]
    iota16 = lax.iota(jnp.int32, _L)
    zero16 = jnp.zeros((_L,), jnp.float32)

    for j in range(nebs):
        s = j % 2
        if j + 1 < nebs:
            _issue_eloads(j + 1, 1 - s)
        _wait_eloads(s)
        if j >= 2:
            pltpu.make_async_copy(ab[s], alpha_hbm.at[pl.ds(0, _E_BLK)],
                                  asem[s]).wait()

        def _alpha(i, _, s=s):
            sl = pl.ds(i * _L, _L)
            d16 = dstb[s][sl]
            dvals = zero16
            for k in range(_L):
                dk = denv[pl.ds(d16[k], _L)]
                dvals = jnp.where(iota16 == k, dk[0], dvals)
            ab[s][sl] = wbb[s][sl] / dvals
            return 0
        lax.fori_loop(0, _E_BLK // _L, _alpha, 0)
        pltpu.async_copy(ab[s], alpha_hbm.at[pl.ds(_eb(j), _E_BLK)], asem[s])
    for j in (nebs - 2, nebs - 1):
        s = j % 2
        pltpu.make_async_copy(ab[s], alpha_hbm.at[pl.ds(0, _E_BLK)],
                              asem[s]).wait()

    for k in range(nkb):
        m = k % 2
        if k >= 1:
            @pl.when(_rbase(k - 1) < _N)
            def _dro(m=m):
                pltpu.make_async_copy(n0[1 - m],
                                      out_hbm.at[pl.ds(0, _NODE_BLK)],
                                      osem[1 - m]).wait()
        if k + 1 < nkb:
            @pl.when(_rbase(k + 1) < _N)
            def _pfn(k=k, m=m):
                _issue_nloads(k + 1, 1 - m)

        @pl.when(_rbase(k) < _N)
        def _blk(k=k, m=m):
            rb = _rbase(k)
            _wait_nloads(m)

            def _rowg(g, _):
                d16 = denv[pl.ds(rb + g * _L, _L)]
                inv16 = 1.0 / jnp.maximum(d16, 1e-30)
                for i in range(_L):
                    r = g * _L + i
                    inv = inv16[i]
                    for cc in range(8):
                        sl = pl.ds(cc * _L, _L)
                        n0[m][r, sl] = ((n0[m][r, sl] + n1[m][r, sl]) * inv
                                        + bias_regs[cc])
                return 0
            lax.fori_loop(0, _NODE_BLK // _L, _rowg, 0)
            pltpu.async_copy(n0[m], out_hbm.at[pl.ds(rb, _NODE_BLK)], osem[m])
    @pl.when(_rbase(nkb - 1) < _N)
    def _drl():
        pltpu.make_async_copy(n0[(nkb - 1) % 2],
                              out_hbm.at[pl.ds(0, _NODE_BLK)],
                              osem[(nkb - 1) % 2]).wait()


_mesh = plsc.VectorSubcoreMesh(core_axis_name="c", subcore_axis_name="s")

_pass1 = pl.kernel(
    _pass1_body,
    [
        jax.ShapeDtypeStruct((2, _NPAD, _C), jnp.float32),
        jax.ShapeDtypeStruct((2, _NPAD), jnp.float32),
        jax.ShapeDtypeStruct((_E,), jnp.float32),
    ],
    mesh=_mesh,
    scratch_types=[
        pltpu.VMEM((_K, _C), jnp.float32),
        pltpu.VMEM((_K, _C), jnp.float32),
        pltpu.VMEM((_K, _C), jnp.float32),
        pltpu.VMEM((_K, _C), jnp.float32),
        pltpu.VMEM((_K,), jnp.int32),
        pltpu.VMEM((_K,), jnp.int32),
        pltpu.VMEM((_K,), jnp.int32),
        pltpu.VMEM((_K,), jnp.int32),
        pltpu.VMEM((_K,), jnp.int32),
        pltpu.VMEM((_K,), jnp.int32),
        pltpu.VMEM((_K,), jnp.int32),
        pltpu.VMEM((_K,), jnp.int32),
        pltpu.VMEM((_K,), jnp.float32),
        pltpu.VMEM((_K,), jnp.float32),
        pltpu.VMEM((_C,), jnp.float32),
        pltpu.VMEM((_ROWS_PT,), jnp.float32),
        pltpu.VMEM((32, _C), jnp.float32),
        pltpu.VMEM_SHARED((_NPAD, _C), jnp.float32),
        pltpu.VMEM_SHARED((_NPAD,), jnp.float32),
        pltpu.SemaphoreType.DMA,
        pltpu.SemaphoreType.DMA,
        pltpu.SemaphoreType.DMA,
        pltpu.SemaphoreType.DMA,
        pltpu.SemaphoreType.DMA,
        pltpu.SemaphoreType.DMA,
        pltpu.SemaphoreType.DMA,
        pltpu.SemaphoreType.DMA,
        pltpu.SemaphoreType.DMA,
        pltpu.SemaphoreType.DMA,
        pltpu.SemaphoreType.DMA,
        pltpu.SemaphoreType.DMA,
    ],
)

_pass2 = pl.kernel(
    _pass2_body,
    [
        jax.ShapeDtypeStruct((_N, _C), jnp.float32),
        jax.ShapeDtypeStruct((_E,), jnp.float32),
    ],
    mesh=_mesh,
    scratch_types=[
        pltpu.VMEM((_NPAD,), jnp.float32),
        pltpu.VMEM((_NPAD,), jnp.float32),
        pltpu.VMEM((_C,), jnp.float32),
        pltpu.VMEM((_E_BLK,), jnp.int32),
        pltpu.VMEM((_E_BLK,), jnp.int32),
        pltpu.VMEM((_E_BLK,), jnp.float32),
        pltpu.VMEM((_E_BLK,), jnp.float32),
        pltpu.VMEM((_E_BLK,), jnp.float32),
        pltpu.VMEM((_E_BLK,), jnp.float32),
        pltpu.VMEM((_NODE_BLK, _C), jnp.float32),
        pltpu.VMEM((_NODE_BLK, _C), jnp.float32),
        pltpu.VMEM((_NODE_BLK, _C), jnp.float32),
        pltpu.VMEM((_NODE_BLK, _C), jnp.float32),
        pltpu.SemaphoreType.DMA,
        pltpu.SemaphoreType.DMA,
        pltpu.SemaphoreType.DMA,
        pltpu.SemaphoreType.DMA,
        pltpu.SemaphoreType.DMA,
        pltpu.SemaphoreType.DMA,
        pltpu.SemaphoreType.DMA,
        pltpu.SemaphoreType.DMA,
        pltpu.SemaphoreType.DMA,
    ],
)


def kernel(x, edge_index, W_l, W_r, att, bias):
    xl, xr = _dense_transforms(x, W_l, W_r)
    eflat = edge_index.reshape(-1)
    num, den, w = _pass1(xl, xr, eflat, att.reshape(-1))
    out, alpha = _pass2(num, den, eflat, w, bias)
    return out, alpha.reshape(_E, 1)
